# baseline (speedup 1.0000x reference)
import functools
import jax
import jax.numpy as jnp
from jax.experimental import pallas as pl
from jax.experimental.pallas import tpu as pltpu

_N = 100000
_C = 1000
_NB = 15
_BLK = 1000


def _ece_kernel(x_ref, lab_ref, out_ref, acc_ref):
    i = pl.program_id(0)
    nsteps = pl.num_programs(0)

    @pl.when(i == 0)
    def _():
        acc_ref[...] = jnp.zeros_like(acc_ref)

    x = x_ref[...]
    lab = lab_ref[...]
    m = jnp.max(x, axis=1, keepdims=True)
    s = jnp.sum(jnp.exp(x - m), axis=1, keepdims=True)
    conf = 1.0 / s
    pred = jnp.argmax(x, axis=1)[:, None].astype(jnp.int32)
    accf = (pred == lab).astype(jnp.float32)
    d = conf - accf
    boundaries = (
        jax.lax.broadcasted_iota(jnp.int32, (1, _NB + 1), 1).astype(jnp.float32)
        / _NB
    )
    mask = (conf > boundaries).astype(jnp.float32)
    acc_ref[...] += jnp.sum(d * mask, axis=0, keepdims=True)

    @pl.when(i == nsteps - 1)
    def _():
        t = acc_ref[...]
        gaps = jnp.abs(t[:, : _NB] - t[:, 1 : _NB + 1])
        out_ref[...] = jnp.sum(gaps, axis=1, keepdims=True) / _N


@functools.partial(jax.jit)
def kernel(logits, labels):
    n = logits.shape[0]
    grid = n // _BLK
    out = pl.pallas_call(
        _ece_kernel,
        grid=(grid,),
        in_specs=[
            pl.BlockSpec((_BLK, _C), lambda i: (i, 0)),
            pl.BlockSpec((_BLK, 1), lambda i: (i, 0)),
        ],
        out_specs=pl.BlockSpec((1, 1), lambda i: (0, 0)),
        out_shape=jax.ShapeDtypeStruct((1, 1), jnp.float32),
        scratch_shapes=[pltpu.VMEM((1, _NB + 1), jnp.float32)],
    )(logits, labels.astype(jnp.int32).reshape(n, 1))
    return out.reshape(1)


# speedup vs baseline: 1.0660x; 1.0660x over previous
import functools
import jax
import jax.numpy as jnp
from jax.experimental import pallas as pl
from jax.experimental.pallas import tpu as pltpu

_N = 100000
_C = 1000
_NB = 15
_BLK = 2000


def _ece_kernel(x_ref, lab_ref, out_ref, acc_ref):
    i = pl.program_id(0)
    nsteps = pl.num_programs(0)

    @pl.when(i == 0)
    def _():
        acc_ref[...] = jnp.zeros_like(acc_ref)

    x = x_ref[...]
    lab = lab_ref[...]
    m = jnp.max(x, axis=1, keepdims=True)
    s = jnp.sum(jnp.exp(x - m), axis=1, keepdims=True)
    conf = 1.0 / s
    pred = jnp.argmax(x, axis=1)[:, None].astype(jnp.int32)
    accf = (pred == lab).astype(jnp.float32)
    d = conf - accf
    boundaries = (
        jax.lax.broadcasted_iota(jnp.int32, (1, _NB + 1), 1).astype(jnp.float32)
        / _NB
    )
    mask = (conf > boundaries).astype(jnp.float32)
    acc_ref[...] += jnp.sum(d * mask, axis=0, keepdims=True)

    @pl.when(i == nsteps - 1)
    def _():
        t = acc_ref[...]
        gaps = jnp.abs(t[:, : _NB] - t[:, 1 : _NB + 1])
        out_ref[...] = jnp.sum(gaps, axis=1, keepdims=True) / _N


@functools.partial(jax.jit)
def kernel(logits, labels):
    n = logits.shape[0]
    grid = n // _BLK
    out = pl.pallas_call(
        _ece_kernel,
        grid=(grid,),
        in_specs=[
            pl.BlockSpec((_BLK, _C), lambda i: (i, 0)),
            pl.BlockSpec((_BLK, 1), lambda i: (i, 0)),
        ],
        out_specs=pl.BlockSpec((1, 1), lambda i: (0, 0)),
        out_shape=jax.ShapeDtypeStruct((1, 1), jnp.float32),
        scratch_shapes=[pltpu.VMEM((1, _NB + 1), jnp.float32)],
    )(logits, labels.astype(jnp.int32).reshape(n, 1))
    return out.reshape(1)


# speedup vs baseline: 1.0909x; 1.0233x over previous
import functools
import jax
import jax.numpy as jnp
from jax.experimental import pallas as pl
from jax.experimental.pallas import tpu as pltpu

_N = 100000
_C = 1000
_NB = 15
_BLK = 4000


def _ece_kernel(x_ref, lab_ref, out_ref, acc_ref):
    i = pl.program_id(0)
    nsteps = pl.num_programs(0)

    @pl.when(i == 0)
    def _():
        acc_ref[...] = jnp.zeros_like(acc_ref)

    x = x_ref[...]
    lab = lab_ref[...]
    m = jnp.max(x, axis=1, keepdims=True)
    s = jnp.sum(jnp.exp(x - m), axis=1, keepdims=True)
    conf = 1.0 / s
    pred = jnp.argmax(x, axis=1)[:, None].astype(jnp.int32)
    accf = (pred == lab).astype(jnp.float32)
    d = conf - accf
    boundaries = (
        jax.lax.broadcasted_iota(jnp.int32, (1, _NB + 1), 1).astype(jnp.float32)
        / _NB
    )
    mask = (conf > boundaries).astype(jnp.float32)
    acc_ref[...] += jnp.sum(d * mask, axis=0, keepdims=True)

    @pl.when(i == nsteps - 1)
    def _():
        t = acc_ref[...]
        gaps = jnp.abs(t[:, : _NB] - t[:, 1 : _NB + 1])
        out_ref[...] = jnp.sum(gaps, axis=1, keepdims=True) / _N


@functools.partial(jax.jit)
def kernel(logits, labels):
    n = logits.shape[0]
    grid = n // _BLK
    out = pl.pallas_call(
        _ece_kernel,
        grid=(grid,),
        in_specs=[
            pl.BlockSpec((_BLK, _C), lambda i: (i, 0)),
            pl.BlockSpec((_BLK, 1), lambda i: (i, 0)),
        ],
        out_specs=pl.BlockSpec((1, 1), lambda i: (0, 0)),
        out_shape=jax.ShapeDtypeStruct((1, 1), jnp.float32),
        scratch_shapes=[pltpu.VMEM((1, _NB + 1), jnp.float32)],
    )(logits, labels.astype(jnp.int32).reshape(n, 1))
    return out.reshape(1)
